# Initial kernel scaffold; baseline (speedup 1.0000x reference)
import jax
import jax.numpy as jnp
from jax.experimental import pallas as pl


def kernel(x, edge_attr, params, edge_index):
    raise NotImplementedError("write your pallas kernel here")


# speedup vs baseline: 1.0454x; 1.0454x over previous
import functools

import jax
import jax.numpy as jnp
from jax import lax
from jax.experimental import pallas as pl
from jax.experimental.pallas import tpu as pltpu

N_NODES = 10000
N_EDGES = 320000
D = 128
STEPS = 5
OUT_SIZE = 3

BE = 3200
BN = 2000

_F32 = jnp.float32


def _dot(a, b):
    return jnp.dot(a, b, preferred_element_type=_F32)


def _ln(o, g, be):
    m = jnp.mean(o, axis=-1, keepdims=True)
    v = jnp.mean((o - m) ** 2, axis=-1, keepdims=True)
    return (o - m) * lax.rsqrt(v + 1e-5) * g + be


def _enc_node_body(x_ref, w_ref, aux_ref, wpre_ref, x0_ref, pa_ref, pb_ref):
    x = x_ref[...]
    h = jnp.maximum(_dot(x, w_ref[0]) + aux_ref[0], 0.0)
    h = jnp.maximum(_dot(h, w_ref[1]) + aux_ref[1], 0.0)
    o = _dot(h, w_ref[2]) + aux_ref[2]
    x0 = _ln(o, aux_ref[3], aux_ref[4])
    x0_ref[...] = x0
    pa_ref[...] = _dot(x0, wpre_ref[0]) + aux_ref[5]
    pb_ref[...] = _dot(x0, wpre_ref[1])


def _edge0_body(ga_ref, gb_ref, ea_ref, ew1_ref, ew_ref, eaux_ref,
                w_ref, aux_ref, ue_ref):
    h = jnp.maximum(_dot(ea_ref[...], ew1_ref[...]) + eaux_ref[0], 0.0)
    h = jnp.maximum(_dot(h, ew_ref[0]) + eaux_ref[1], 0.0)
    o = _dot(h, ew_ref[1]) + eaux_ref[2]
    e = _ln(o, eaux_ref[3], eaux_ref[4])
    h = jnp.maximum(ga_ref[...] + gb_ref[...] + _dot(e, w_ref[0]), 0.0)
    h = jnp.maximum(_dot(h, w_ref[1]) + aux_ref[0], 0.0)
    o = _dot(h, w_ref[2]) + aux_ref[1]
    ue_ref[...] = _ln(o, aux_ref[2], aux_ref[3])


def _edge_body(ga_ref, gb_ref, e_ref, w_ref, aux_ref, ue_ref):
    h = jnp.maximum(ga_ref[...] + gb_ref[...] + _dot(e_ref[...], w_ref[0]), 0.0)
    h = jnp.maximum(_dot(h, w_ref[1]) + aux_ref[0], 0.0)
    o = _dot(h, w_ref[2]) + aux_ref[1]
    ue_ref[...] = _ln(o, aux_ref[2], aux_ref[3])


def _node_body(x_ref, agg_ref, w_ref, aux_ref, wpre_ref,
               x1_ref, pa_ref, pb_ref):
    h = jnp.maximum(_dot(x_ref[...], w_ref[0]) + _dot(agg_ref[...], w_ref[1])
                    + aux_ref[0], 0.0)
    h = jnp.maximum(_dot(h, w_ref[2]) + aux_ref[1], 0.0)
    o = _dot(h, w_ref[3]) + aux_ref[2]
    x1 = _ln(o, aux_ref[3], aux_ref[4])
    x1_ref[...] = x1
    pa_ref[...] = _dot(x1, wpre_ref[0]) + aux_ref[5]
    pb_ref[...] = _dot(x1, wpre_ref[1])


def _node_dec_body(x_ref, agg_ref, w_ref, aux_ref, wd_ref, daux_ref, out_ref):
    h = jnp.maximum(_dot(x_ref[...], w_ref[0]) + _dot(agg_ref[...], w_ref[1])
                    + aux_ref[0], 0.0)
    h = jnp.maximum(_dot(h, w_ref[2]) + aux_ref[1], 0.0)
    o = _dot(h, w_ref[3]) + aux_ref[2]
    x1 = _ln(o, aux_ref[3], aux_ref[4])
    h = jnp.maximum(_dot(x1, wd_ref[0]) + daux_ref[0], 0.0)
    h = jnp.maximum(_dot(h, wd_ref[1]) + daux_ref[1], 0.0)
    out_ref[...] = _dot(h, wd_ref[2]) + daux_ref[2]


def _full3(a):
    return pl.BlockSpec(a.shape, lambda i: (0,) * a.ndim)


def _rows(block_rows, ncols):
    return pl.BlockSpec((block_rows, ncols), lambda i: (i, 0))


def _enc_node(x, w, aux, wpre):
    grid = (N_NODES // BN,)
    out = jax.ShapeDtypeStruct((N_NODES, D), _F32)
    return pl.pallas_call(
        _enc_node_body,
        grid=grid,
        in_specs=[_rows(BN, D), _full3(w), _full3(aux), _full3(wpre)],
        out_specs=[_rows(BN, D)] * 3,
        out_shape=[out, out, out],
    )(x, w, aux, wpre)


def _edge0(ga, gb, ea, ew1, ew, eaux, w, aux):
    grid = (N_EDGES // BE,)
    return pl.pallas_call(
        _edge0_body,
        grid=grid,
        in_specs=[_rows(BE, D), _rows(BE, D), _rows(BE, ea.shape[1]),
                  _full3(ew1), _full3(ew), _full3(eaux), _full3(w), _full3(aux)],
        out_specs=_rows(BE, D),
        out_shape=jax.ShapeDtypeStruct((N_EDGES, D), _F32),
    )(ga, gb, ea, ew1, ew, eaux, w, aux)


def _edge(ga, gb, e, w, aux):
    grid = (N_EDGES // BE,)
    return pl.pallas_call(
        _edge_body,
        grid=grid,
        in_specs=[_rows(BE, D), _rows(BE, D), _rows(BE, D), _full3(w), _full3(aux)],
        out_specs=_rows(BE, D),
        out_shape=jax.ShapeDtypeStruct((N_EDGES, D), _F32),
    )(ga, gb, e, w, aux)


def _node(x, agg, w, aux, wpre):
    grid = (N_NODES // BN,)
    out = jax.ShapeDtypeStruct((N_NODES, D), _F32)
    return pl.pallas_call(
        _node_body,
        grid=grid,
        in_specs=[_rows(BN, D), _rows(BN, D), _full3(w), _full3(aux), _full3(wpre)],
        out_specs=[_rows(BN, D)] * 3,
        out_shape=[out, out, out],
    )(x, agg, w, aux, wpre)


def _node_dec(x, agg, w, aux, wd, daux):
    grid = (N_NODES // BN,)
    return pl.pallas_call(
        _node_dec_body,
        grid=grid,
        in_specs=[_rows(BN, D), _rows(BN, D), _full3(w), _full3(aux),
                  _full3(wd), _full3(daux)],
        out_specs=_rows(BN, D),
        out_shape=jax.ShapeDtypeStruct((N_NODES, D), _F32),
    )(x, agg, w, aux, wd, daux)


def _aux(*rows):
    a = jnp.zeros((8, D), _F32)
    for i, r in enumerate(rows):
        a = a.at[i].set(r)
    return a


def _pack(params):
    p = {}
    en = params['enc_node']['mlp']
    pe0 = params['proc'][0]['edge']['mlp']
    p['enc_w'] = jnp.stack([en[0]['W'], en[1]['W'], en[2]['W']])
    p['enc_wpre'] = jnp.stack([pe0[0]['W'][:D], pe0[0]['W'][D:2 * D]])
    p['enc_aux'] = _aux(en[0]['b'], en[1]['b'], en[2]['b'],
                        params['enc_node']['g'], params['enc_node']['be'],
                        pe0[0]['b'])
    ee = params['enc_edge']['mlp']
    p['ee_w1'] = ee[0]['W']
    p['ee_w'] = jnp.stack([ee[1]['W'], ee[2]['W']])
    p['ee_aux'] = _aux(ee[0]['b'], ee[1]['b'], ee[2]['b'],
                       params['enc_edge']['g'], params['enc_edge']['be'])
    p['edge'] = []
    p['node'] = []
    for i in range(STEPS):
        pe = params['proc'][i]['edge']
        pn = params['proc'][i]['node']
        p['edge'].append({
            'w': jnp.stack([pe['mlp'][0]['W'][2 * D:], pe['mlp'][1]['W'],
                            pe['mlp'][2]['W']]),
            'aux': _aux(pe['mlp'][1]['b'], pe['mlp'][2]['b'], pe['g'], pe['be']),
        })
        nd = {
            'w': jnp.stack([pn['mlp'][0]['W'][:D], pn['mlp'][0]['W'][D:],
                            pn['mlp'][1]['W'], pn['mlp'][2]['W']]),
        }
        if i + 1 < STEPS:
            pe1 = params['proc'][i + 1]['edge']['mlp']
            nd['wpre'] = jnp.stack([pe1[0]['W'][:D], pe1[0]['W'][D:2 * D]])
            nd['aux'] = _aux(pn['mlp'][0]['b'], pn['mlp'][1]['b'], pn['mlp'][2]['b'],
                             pn['g'], pn['be'], pe1[0]['b'])
        else:
            nd['aux'] = _aux(pn['mlp'][0]['b'], pn['mlp'][1]['b'], pn['mlp'][2]['b'],
                             pn['g'], pn['be'])
        p['node'].append(nd)
    dc = params['dec']
    w3 = jnp.zeros((D, D), _F32).at[:, :OUT_SIZE].set(dc[2]['W'])
    b3 = jnp.zeros((D,), _F32).at[:OUT_SIZE].set(dc[2]['b'])
    p['dec_w'] = jnp.stack([dc[0]['W'], dc[1]['W'], w3])
    p['dec_aux'] = _aux(dc[0]['b'], dc[1]['b'], b3)
    return p


def _gather(pa, pb, s, r):
    return jnp.take(pa, s, axis=0), jnp.take(pb, r, axis=0)


def _segment_sum(ue, r):
    return jax.ops.segment_sum(ue, r, num_segments=N_NODES)


def kernel(x, edge_attr, params, edge_index):
    p = _pack(params)
    s = edge_index[0]
    r = edge_index[1]
    x0, pa, pb = _enc_node(x, p['enc_w'], p['enc_aux'], p['enc_wpre'])
    xc = x0
    e = None
    for i in range(STEPS):
        ga, gb = _gather(pa, pb, s, r)
        if i == 0:
            ue = _edge0(ga, gb, edge_attr, p['ee_w1'], p['ee_w'], p['ee_aux'],
                        p['edge'][0]['w'], p['edge'][0]['aux'])
        else:
            ue = _edge(ga, gb, e, p['edge'][i]['w'], p['edge'][i]['aux'])
        agg = _segment_sum(ue, r)
        if i + 1 < STEPS:
            xc, pa, pb = _node(xc, agg, p['node'][i]['w'], p['node'][i]['aux'],
                               p['node'][i]['wpre'])
            e = ue
        else:
            out = _node_dec(xc, agg, p['node'][i]['w'], p['node'][i]['aux'],
                            p['dec_w'], p['dec_aux'])
    return out[:, :OUT_SIZE]


# speedup vs baseline: 1.9553x; 1.8704x over previous
import functools

import jax
import jax.numpy as jnp
from jax import lax
from jax.experimental import pallas as pl
from jax.experimental.pallas import tpu as pltpu
from jax.experimental.pallas import tpu_sc as plsc

N_NODES = 10000
N_EDGES = 320000
D = 128
STEPS = 5
OUT_SIZE = 3

BE = 3200
BN = 2000

_F32 = jnp.float32


def _dot(a, b):
    return jnp.dot(a, b, preferred_element_type=_F32)


def _ln(o, g, be):
    m = jnp.mean(o, axis=-1, keepdims=True)
    v = jnp.mean((o - m) ** 2, axis=-1, keepdims=True)
    return (o - m) * lax.rsqrt(v + 1e-5) * g + be


def _enc_node_body(x_ref, w_ref, aux_ref, wpre_ref, x0_ref, pa_ref, pb_ref):
    x = x_ref[...]
    h = jnp.maximum(_dot(x, w_ref[0]) + aux_ref[0], 0.0)
    h = jnp.maximum(_dot(h, w_ref[1]) + aux_ref[1], 0.0)
    o = _dot(h, w_ref[2]) + aux_ref[2]
    x0 = _ln(o, aux_ref[3], aux_ref[4])
    x0_ref[...] = x0
    pa_ref[...] = _dot(x0, wpre_ref[0]) + aux_ref[5]
    pb_ref[...] = _dot(x0, wpre_ref[1])


def _edge0_body(ga_ref, gb_ref, ea_ref, ew1_ref, ew_ref, eaux_ref,
                w_ref, aux_ref, ue_ref):
    h = jnp.maximum(_dot(ea_ref[...], ew1_ref[...]) + eaux_ref[0], 0.0)
    h = jnp.maximum(_dot(h, ew_ref[0]) + eaux_ref[1], 0.0)
    o = _dot(h, ew_ref[1]) + eaux_ref[2]
    e = _ln(o, eaux_ref[3], eaux_ref[4])
    h = jnp.maximum(ga_ref[...] + gb_ref[...] + _dot(e, w_ref[0]), 0.0)
    h = jnp.maximum(_dot(h, w_ref[1]) + aux_ref[0], 0.0)
    o = _dot(h, w_ref[2]) + aux_ref[1]
    ue_ref[...] = _ln(o, aux_ref[2], aux_ref[3])


def _edge_body(ga_ref, gb_ref, e_ref, w_ref, aux_ref, ue_ref):
    h = jnp.maximum(ga_ref[...] + gb_ref[...] + _dot(e_ref[...], w_ref[0]), 0.0)
    h = jnp.maximum(_dot(h, w_ref[1]) + aux_ref[0], 0.0)
    o = _dot(h, w_ref[2]) + aux_ref[1]
    ue_ref[...] = _ln(o, aux_ref[2], aux_ref[3])


def _node_body(x_ref, agg_ref, w_ref, aux_ref, wpre_ref,
               x1_ref, pa_ref, pb_ref):
    h = jnp.maximum(_dot(x_ref[...], w_ref[0]) + _dot(agg_ref[...], w_ref[1])
                    + aux_ref[0], 0.0)
    h = jnp.maximum(_dot(h, w_ref[2]) + aux_ref[1], 0.0)
    o = _dot(h, w_ref[3]) + aux_ref[2]
    x1 = _ln(o, aux_ref[3], aux_ref[4])
    x1_ref[...] = x1
    pa_ref[...] = _dot(x1, wpre_ref[0]) + aux_ref[5]
    pb_ref[...] = _dot(x1, wpre_ref[1])


def _node_dec_body(x_ref, agg_ref, w_ref, aux_ref, wd_ref, daux_ref, out_ref):
    h = jnp.maximum(_dot(x_ref[...], w_ref[0]) + _dot(agg_ref[...], w_ref[1])
                    + aux_ref[0], 0.0)
    h = jnp.maximum(_dot(h, w_ref[2]) + aux_ref[1], 0.0)
    o = _dot(h, w_ref[3]) + aux_ref[2]
    x1 = _ln(o, aux_ref[3], aux_ref[4])
    h = jnp.maximum(_dot(x1, wd_ref[0]) + daux_ref[0], 0.0)
    h = jnp.maximum(_dot(h, wd_ref[1]) + daux_ref[1], 0.0)
    out_ref[...] = _dot(h, wd_ref[2]) + daux_ref[2]


def _full3(a):
    return pl.BlockSpec(a.shape, lambda i: (0,) * a.ndim)


def _rows(block_rows, ncols):
    return pl.BlockSpec((block_rows, ncols), lambda i: (i, 0))


def _enc_node(x, w, aux, wpre):
    grid = (N_NODES // BN,)
    out = jax.ShapeDtypeStruct((N_NODES, D), _F32)
    return pl.pallas_call(
        _enc_node_body,
        grid=grid,
        in_specs=[_rows(BN, D), _full3(w), _full3(aux), _full3(wpre)],
        out_specs=[_rows(BN, D)] * 3,
        out_shape=[out, out, out],
    )(x, w, aux, wpre)


def _edge0(ga, gb, ea, ew1, ew, eaux, w, aux):
    grid = (N_EDGES // BE,)
    return pl.pallas_call(
        _edge0_body,
        grid=grid,
        in_specs=[_rows(BE, D), _rows(BE, D), _rows(BE, ea.shape[1]),
                  _full3(ew1), _full3(ew), _full3(eaux), _full3(w), _full3(aux)],
        out_specs=_rows(BE, D),
        out_shape=jax.ShapeDtypeStruct((N_EDGES, D), _F32),
    )(ga, gb, ea, ew1, ew, eaux, w, aux)


def _edge(ga, gb, e, w, aux):
    grid = (N_EDGES // BE,)
    return pl.pallas_call(
        _edge_body,
        grid=grid,
        in_specs=[_rows(BE, D), _rows(BE, D), _rows(BE, D), _full3(w), _full3(aux)],
        out_specs=_rows(BE, D),
        out_shape=jax.ShapeDtypeStruct((N_EDGES, D), _F32),
    )(ga, gb, e, w, aux)


def _node(x, agg, w, aux, wpre):
    grid = (N_NODES // BN,)
    out = jax.ShapeDtypeStruct((N_NODES, D), _F32)
    return pl.pallas_call(
        _node_body,
        grid=grid,
        in_specs=[_rows(BN, D), _rows(BN, D), _full3(w), _full3(aux), _full3(wpre)],
        out_specs=[_rows(BN, D)] * 3,
        out_shape=[out, out, out],
    )(x, agg, w, aux, wpre)


def _node_dec(x, agg, w, aux, wd, daux):
    grid = (N_NODES // BN,)
    return pl.pallas_call(
        _node_dec_body,
        grid=grid,
        in_specs=[_rows(BN, D), _rows(BN, D), _full3(w), _full3(aux),
                  _full3(wd), _full3(daux)],
        out_specs=_rows(BN, D),
        out_shape=jax.ShapeDtypeStruct((N_NODES, D), _F32),
    )(x, agg, w, aux, wd, daux)


def _aux(*rows):
    a = jnp.zeros((8, D), _F32)
    for i, r in enumerate(rows):
        a = a.at[i].set(r)
    return a


def _pack(params):
    p = {}
    en = params['enc_node']['mlp']
    pe0 = params['proc'][0]['edge']['mlp']
    p['enc_w'] = jnp.stack([en[0]['W'], en[1]['W'], en[2]['W']])
    p['enc_wpre'] = jnp.stack([pe0[0]['W'][:D], pe0[0]['W'][D:2 * D]])
    p['enc_aux'] = _aux(en[0]['b'], en[1]['b'], en[2]['b'],
                        params['enc_node']['g'], params['enc_node']['be'],
                        pe0[0]['b'])
    ee = params['enc_edge']['mlp']
    p['ee_w1'] = ee[0]['W']
    p['ee_w'] = jnp.stack([ee[1]['W'], ee[2]['W']])
    p['ee_aux'] = _aux(ee[0]['b'], ee[1]['b'], ee[2]['b'],
                       params['enc_edge']['g'], params['enc_edge']['be'])
    p['edge'] = []
    p['node'] = []
    for i in range(STEPS):
        pe = params['proc'][i]['edge']
        pn = params['proc'][i]['node']
        p['edge'].append({
            'w': jnp.stack([pe['mlp'][0]['W'][2 * D:], pe['mlp'][1]['W'],
                            pe['mlp'][2]['W']]),
            'aux': _aux(pe['mlp'][1]['b'], pe['mlp'][2]['b'], pe['g'], pe['be']),
        })
        nd = {
            'w': jnp.stack([pn['mlp'][0]['W'][:D], pn['mlp'][0]['W'][D:],
                            pn['mlp'][1]['W'], pn['mlp'][2]['W']]),
        }
        if i + 1 < STEPS:
            pe1 = params['proc'][i + 1]['edge']['mlp']
            nd['wpre'] = jnp.stack([pe1[0]['W'][:D], pe1[0]['W'][D:2 * D]])
            nd['aux'] = _aux(pn['mlp'][0]['b'], pn['mlp'][1]['b'], pn['mlp'][2]['b'],
                             pn['g'], pn['be'], pe1[0]['b'])
        else:
            nd['aux'] = _aux(pn['mlp'][0]['b'], pn['mlp'][1]['b'], pn['mlp'][2]['b'],
                             pn['g'], pn['be'])
        p['node'].append(nd)
    dc = params['dec']
    w3 = jnp.zeros((D, D), _F32).at[:, :OUT_SIZE].set(dc[2]['W'])
    b3 = jnp.zeros((D,), _F32).at[:OUT_SIZE].set(dc[2]['b'])
    p['dec_w'] = jnp.stack([dc[0]['W'], dc[1]['W'], w3])
    p['dec_aux'] = _aux(dc[0]['b'], dc[1]['b'], b3)
    return p


_NC = 2
_NS = 16
_NW = _NC * _NS
_EPW = N_EDGES // _NW
_CH = 80
_NCH = _EPW // _CH

_sc_mesh = plsc.VectorSubcoreMesh(core_axis_name="c", subcore_axis_name="s")


@functools.partial(
    pl.kernel, mesh=_sc_mesh,
    out_type=[jax.ShapeDtypeStruct((N_EDGES, D), _F32),
              jax.ShapeDtypeStruct((N_EDGES, D), _F32)],
    scratch_types=[pltpu.VMEM((_CH,), jnp.int32), pltpu.VMEM((_CH,), jnp.int32),
                   pltpu.VMEM((_CH, D), _F32), pltpu.VMEM((_CH, D), _F32),
                   pltpu.SemaphoreType.DMA, pltpu.SemaphoreType.DMA],
)
def _sc_gather(pa_hbm, pb_hbm, s_hbm, r_hbm, ga_hbm, gb_hbm,
               sidx, ridx, bufa, bufb, sema, semb):
    wid = lax.axis_index("s") * _NC + lax.axis_index("c")
    base = wid * _EPW

    def body(i, carry):
        off = base + i * _CH
        pltpu.sync_copy(s_hbm.at[pl.ds(off, _CH)], sidx)
        pltpu.sync_copy(r_hbm.at[pl.ds(off, _CH)], ridx)
        ca = pltpu.async_copy(pa_hbm.at[sidx], bufa, sema)
        cb = pltpu.async_copy(pb_hbm.at[ridx], bufb, semb)
        ca.wait()
        cb.wait()
        pltpu.sync_copy(bufa, ga_hbm.at[pl.ds(off, _CH)])
        pltpu.sync_copy(bufb, gb_hbm.at[pl.ds(off, _CH)])
        return carry

    lax.fori_loop(0, _NCH, body, 0)


def _gather(pa, pb, s, r):
    return _sc_gather(pa, pb, s, r)


def _segment_sum(ue, r):
    return jax.ops.segment_sum(ue, r, num_segments=N_NODES)


def kernel(x, edge_attr, params, edge_index):
    p = _pack(params)
    s = edge_index[0]
    r = edge_index[1]
    x0, pa, pb = _enc_node(x, p['enc_w'], p['enc_aux'], p['enc_wpre'])
    xc = x0
    e = None
    for i in range(STEPS):
        ga, gb = _gather(pa, pb, s, r)
        if i == 0:
            ue = _edge0(ga, gb, edge_attr, p['ee_w1'], p['ee_w'], p['ee_aux'],
                        p['edge'][0]['w'], p['edge'][0]['aux'])
        else:
            ue = _edge(ga, gb, e, p['edge'][i]['w'], p['edge'][i]['aux'])
        agg = _segment_sum(ue, r)
        if i + 1 < STEPS:
            xc, pa, pb = _node(xc, agg, p['node'][i]['w'], p['node'][i]['aux'],
                               p['node'][i]['wpre'])
            e = ue
        else:
            out = _node_dec(xc, agg, p['node'][i]['w'], p['node'][i]['aux'],
                            p['dec_w'], p['dec_aux'])
    return out[:, :OUT_SIZE]


# speedup vs baseline: 2.8382x; 1.4516x over previous
import functools

import jax
import jax.numpy as jnp
from jax import lax
from jax.experimental import pallas as pl
from jax.experimental.pallas import tpu as pltpu
from jax.experimental.pallas import tpu_sc as plsc

N_NODES = 10000
N_EDGES = 320000
D = 128
STEPS = 5
OUT_SIZE = 3

BE = 3200
BN = 2000

_F32 = jnp.float32


def _dot(a, b):
    return jnp.dot(a, b, preferred_element_type=_F32)


def _ln(o, g, be):
    m = jnp.mean(o, axis=-1, keepdims=True)
    v = jnp.mean((o - m) ** 2, axis=-1, keepdims=True)
    return (o - m) * lax.rsqrt(v + 1e-5) * g + be


def _enc_node_body(x_ref, w_ref, aux_ref, wpre_ref, x0_ref, pa_ref, pb_ref):
    x = x_ref[...]
    h = jnp.maximum(_dot(x, w_ref[0]) + aux_ref[0], 0.0)
    h = jnp.maximum(_dot(h, w_ref[1]) + aux_ref[1], 0.0)
    o = _dot(h, w_ref[2]) + aux_ref[2]
    x0 = _ln(o, aux_ref[3], aux_ref[4])
    x0_ref[...] = x0
    pa_ref[...] = _dot(x0, wpre_ref[0]) + aux_ref[5]
    pb_ref[...] = _dot(x0, wpre_ref[1])


def _edge0_body(ga_ref, gb_ref, ea_ref, ew1_ref, ew_ref, eaux_ref,
                w_ref, aux_ref, ue_ref):
    h = jnp.maximum(_dot(ea_ref[...], ew1_ref[...]) + eaux_ref[0], 0.0)
    h = jnp.maximum(_dot(h, ew_ref[0]) + eaux_ref[1], 0.0)
    o = _dot(h, ew_ref[1]) + eaux_ref[2]
    e = _ln(o, eaux_ref[3], eaux_ref[4])
    h = jnp.maximum(ga_ref[...] + gb_ref[...] + _dot(e, w_ref[0]), 0.0)
    h = jnp.maximum(_dot(h, w_ref[1]) + aux_ref[0], 0.0)
    o = _dot(h, w_ref[2]) + aux_ref[1]
    ue_ref[...] = _ln(o, aux_ref[2], aux_ref[3])


def _edge_body(ga_ref, gb_ref, e_ref, w_ref, aux_ref, ue_ref):
    h = jnp.maximum(ga_ref[...] + gb_ref[...] + _dot(e_ref[...], w_ref[0]), 0.0)
    h = jnp.maximum(_dot(h, w_ref[1]) + aux_ref[0], 0.0)
    o = _dot(h, w_ref[2]) + aux_ref[1]
    ue_ref[...] = _ln(o, aux_ref[2], aux_ref[3])


def _node_body(x_ref, agg_ref, w_ref, aux_ref, wpre_ref,
               x1_ref, pa_ref, pb_ref):
    h = jnp.maximum(_dot(x_ref[...], w_ref[0]) + _dot(agg_ref[...], w_ref[1])
                    + aux_ref[0], 0.0)
    h = jnp.maximum(_dot(h, w_ref[2]) + aux_ref[1], 0.0)
    o = _dot(h, w_ref[3]) + aux_ref[2]
    x1 = _ln(o, aux_ref[3], aux_ref[4])
    x1_ref[...] = x1
    pa_ref[...] = _dot(x1, wpre_ref[0]) + aux_ref[5]
    pb_ref[...] = _dot(x1, wpre_ref[1])


def _node_dec_body(x_ref, agg_ref, w_ref, aux_ref, wd_ref, daux_ref, out_ref):
    h = jnp.maximum(_dot(x_ref[...], w_ref[0]) + _dot(agg_ref[...], w_ref[1])
                    + aux_ref[0], 0.0)
    h = jnp.maximum(_dot(h, w_ref[2]) + aux_ref[1], 0.0)
    o = _dot(h, w_ref[3]) + aux_ref[2]
    x1 = _ln(o, aux_ref[3], aux_ref[4])
    h = jnp.maximum(_dot(x1, wd_ref[0]) + daux_ref[0], 0.0)
    h = jnp.maximum(_dot(h, wd_ref[1]) + daux_ref[1], 0.0)
    out_ref[...] = _dot(h, wd_ref[2]) + daux_ref[2]


def _full3(a):
    return pl.BlockSpec(a.shape, lambda i: (0,) * a.ndim)


def _rows(block_rows, ncols):
    return pl.BlockSpec((block_rows, ncols), lambda i: (i, 0))


def _enc_node(x, w, aux, wpre):
    grid = (N_NODES // BN,)
    out = jax.ShapeDtypeStruct((N_NODES, D), _F32)
    return pl.pallas_call(
        _enc_node_body,
        grid=grid,
        in_specs=[_rows(BN, D), _full3(w), _full3(aux), _full3(wpre)],
        out_specs=[_rows(BN, D)] * 3,
        out_shape=[out, out, out],
    )(x, w, aux, wpre)


def _edge0(ga, gb, ea, ew1, ew, eaux, w, aux):
    grid = (N_EDGES // BE,)
    return pl.pallas_call(
        _edge0_body,
        grid=grid,
        in_specs=[_rows(BE, D), _rows(BE, D), _rows(BE, ea.shape[1]),
                  _full3(ew1), _full3(ew), _full3(eaux), _full3(w), _full3(aux)],
        out_specs=_rows(BE, D),
        out_shape=jax.ShapeDtypeStruct((N_EDGES, D), _F32),
    )(ga, gb, ea, ew1, ew, eaux, w, aux)


def _edge(ga, gb, e, w, aux):
    grid = (N_EDGES // BE,)
    return pl.pallas_call(
        _edge_body,
        grid=grid,
        in_specs=[_rows(BE, D), _rows(BE, D), _rows(BE, D), _full3(w), _full3(aux)],
        out_specs=_rows(BE, D),
        out_shape=jax.ShapeDtypeStruct((N_EDGES, D), _F32),
    )(ga, gb, e, w, aux)


def _node(x, agg, w, aux, wpre):
    grid = (N_NODES // BN,)
    out = jax.ShapeDtypeStruct((N_NODES, D), _F32)
    return pl.pallas_call(
        _node_body,
        grid=grid,
        in_specs=[_rows(BN, D), _rows(BN, D), _full3(w), _full3(aux), _full3(wpre)],
        out_specs=[_rows(BN, D)] * 3,
        out_shape=[out, out, out],
    )(x, agg, w, aux, wpre)


def _node_dec(x, agg, w, aux, wd, daux):
    grid = (N_NODES // BN,)
    return pl.pallas_call(
        _node_dec_body,
        grid=grid,
        in_specs=[_rows(BN, D), _rows(BN, D), _full3(w), _full3(aux),
                  _full3(wd), _full3(daux)],
        out_specs=_rows(BN, D),
        out_shape=jax.ShapeDtypeStruct((N_NODES, D), _F32),
    )(x, agg, w, aux, wd, daux)


def _aux(*rows):
    a = jnp.zeros((8, D), _F32)
    for i, r in enumerate(rows):
        a = a.at[i].set(r)
    return a


def _pack(params):
    p = {}
    en = params['enc_node']['mlp']
    pe0 = params['proc'][0]['edge']['mlp']
    p['enc_w'] = jnp.stack([en[0]['W'], en[1]['W'], en[2]['W']])
    p['enc_wpre'] = jnp.stack([pe0[0]['W'][:D], pe0[0]['W'][D:2 * D]])
    p['enc_aux'] = _aux(en[0]['b'], en[1]['b'], en[2]['b'],
                        params['enc_node']['g'], params['enc_node']['be'],
                        pe0[0]['b'])
    ee = params['enc_edge']['mlp']
    p['ee_w1'] = ee[0]['W']
    p['ee_w'] = jnp.stack([ee[1]['W'], ee[2]['W']])
    p['ee_aux'] = _aux(ee[0]['b'], ee[1]['b'], ee[2]['b'],
                       params['enc_edge']['g'], params['enc_edge']['be'])
    p['edge'] = []
    p['node'] = []
    for i in range(STEPS):
        pe = params['proc'][i]['edge']
        pn = params['proc'][i]['node']
        p['edge'].append({
            'w': jnp.stack([pe['mlp'][0]['W'][2 * D:], pe['mlp'][1]['W'],
                            pe['mlp'][2]['W']]),
            'aux': _aux(pe['mlp'][1]['b'], pe['mlp'][2]['b'], pe['g'], pe['be']),
        })
        nd = {
            'w': jnp.stack([pn['mlp'][0]['W'][:D], pn['mlp'][0]['W'][D:],
                            pn['mlp'][1]['W'], pn['mlp'][2]['W']]),
        }
        if i + 1 < STEPS:
            pe1 = params['proc'][i + 1]['edge']['mlp']
            nd['wpre'] = jnp.stack([pe1[0]['W'][:D], pe1[0]['W'][D:2 * D]])
            nd['aux'] = _aux(pn['mlp'][0]['b'], pn['mlp'][1]['b'], pn['mlp'][2]['b'],
                             pn['g'], pn['be'], pe1[0]['b'])
        else:
            nd['aux'] = _aux(pn['mlp'][0]['b'], pn['mlp'][1]['b'], pn['mlp'][2]['b'],
                             pn['g'], pn['be'])
        p['node'].append(nd)
    dc = params['dec']
    w3 = jnp.zeros((D, D), _F32).at[:, :OUT_SIZE].set(dc[2]['W'])
    b3 = jnp.zeros((D,), _F32).at[:OUT_SIZE].set(dc[2]['b'])
    p['dec_w'] = jnp.stack([dc[0]['W'], dc[1]['W'], w3])
    p['dec_aux'] = _aux(dc[0]['b'], dc[1]['b'], b3)
    return p


_NC = 2
_NS = 16
_NW = _NC * _NS
_EPW = N_EDGES // _NW
_CH = 80
_NCH = _EPW // _CH

_sc_mesh = plsc.VectorSubcoreMesh(core_axis_name="c", subcore_axis_name="s")


@functools.partial(
    pl.kernel, mesh=_sc_mesh,
    out_type=[jax.ShapeDtypeStruct((N_EDGES, D), _F32),
              jax.ShapeDtypeStruct((N_EDGES, D), _F32)],
    scratch_types=[pltpu.VMEM((_CH,), jnp.int32), pltpu.VMEM((_CH,), jnp.int32),
                   pltpu.VMEM((_CH, D), _F32), pltpu.VMEM((_CH, D), _F32),
                   pltpu.SemaphoreType.DMA, pltpu.SemaphoreType.DMA],
)
def _sc_gather(pa_hbm, pb_hbm, s_hbm, r_hbm, ga_hbm, gb_hbm,
               sidx, ridx, bufa, bufb, sema, semb):
    wid = lax.axis_index("s") * _NC + lax.axis_index("c")
    base = wid * _EPW

    def body(i, carry):
        off = base + i * _CH
        pltpu.sync_copy(s_hbm.at[pl.ds(off, _CH)], sidx)
        pltpu.sync_copy(r_hbm.at[pl.ds(off, _CH)], ridx)
        ca = pltpu.async_copy(pa_hbm.at[sidx], bufa, sema)
        cb = pltpu.async_copy(pb_hbm.at[ridx], bufb, semb)
        ca.wait()
        cb.wait()
        pltpu.sync_copy(bufa, ga_hbm.at[pl.ds(off, _CH)])
        pltpu.sync_copy(bufb, gb_hbm.at[pl.ds(off, _CH)])
        return carry

    lax.fori_loop(0, _NCH, body, 0)


def _gather(pa, pb, s, r):
    return _sc_gather(pa, pb, s, r)


_NP = 10240
_RPS = _NP // _NS
_ZR = 128


@functools.partial(
    pl.kernel, mesh=_sc_mesh,
    out_type=jax.ShapeDtypeStruct((_NC, _NP, D), _F32),
    scratch_types=[pltpu.VMEM((_CH,), jnp.int32), pltpu.VMEM((_CH, D), _F32),
                   pltpu.VMEM_SHARED((_NP, D), _F32)],
)
def _sc_scatter(ue_hbm, r_hbm, z_hbm, out_hbm, ridx, buf, acc_sh):
    cid = lax.axis_index("c")
    sid = lax.axis_index("s")
    wid = sid * _NC + cid
    rbase = sid * _RPS

    def zs(i, c):
        pltpu.sync_copy(z_hbm, acc_sh.at[pl.ds(rbase + i * _ZR, _ZR)])
        return c

    lax.fori_loop(0, _RPS // _ZR, zs, 0)
    plsc.subcore_barrier()
    base = wid * _EPW

    def body(i, c):
        off = base + i * _CH
        pltpu.sync_copy(r_hbm.at[pl.ds(off, _CH)], ridx)
        pltpu.sync_copy(ue_hbm.at[pl.ds(off, _CH)], buf)
        pltpu.sync_copy(buf, acc_sh.at[ridx], add=True)
        return c

    lax.fori_loop(0, _NCH, body, 0)
    plsc.subcore_barrier()
    pltpu.sync_copy(acc_sh.at[pl.ds(rbase, _RPS)],
                    out_hbm.at[cid, pl.ds(rbase, _RPS)])


def _segment_sum(ue, r):
    z = jnp.zeros((_ZR, D), _F32)
    parts = _sc_scatter(ue, r, z)
    return parts[0, :N_NODES] + parts[1, :N_NODES]


def kernel(x, edge_attr, params, edge_index):
    p = _pack(params)
    s = edge_index[0]
    r = edge_index[1]
    x0, pa, pb = _enc_node(x, p['enc_w'], p['enc_aux'], p['enc_wpre'])
    xc = x0
    e = None
    for i in range(STEPS):
        ga, gb = _gather(pa, pb, s, r)
        if i == 0:
            ue = _edge0(ga, gb, edge_attr, p['ee_w1'], p['ee_w'], p['ee_aux'],
                        p['edge'][0]['w'], p['edge'][0]['aux'])
        else:
            ue = _edge(ga, gb, e, p['edge'][i]['w'], p['edge'][i]['aux'])
        agg = _segment_sum(ue, r)
        if i + 1 < STEPS:
            xc, pa, pb = _node(xc, agg, p['node'][i]['w'], p['node'][i]['aux'],
                               p['node'][i]['wpre'])
            e = ue
        else:
            out = _node_dec(xc, agg, p['node'][i]['w'], p['node'][i]['aux'],
                            p['dec_w'], p['dec_aux'])
    return out[:, :OUT_SIZE]
